# baseline (speedup 1.0000x reference)
import functools

import jax
import jax.numpy as jnp
from jax import lax
from jax.experimental import pallas as pl
from jax.experimental.pallas import tpu as pltpu

N = 50000
E = 800000
IN = 4
H = 256
OUT = 4

BLK = 2000
GRID = N // BLK


def _tc_layer1_pre(s0, s1, d0, d1, x, wl_t, wr_t, b1):
    def body(s0_r, s1_r, d0_r, d1_r, x_r, wl_r, wr_r, b_r, pre_r, st_r):
        i = pl.program_id(0)
        ssum = s0_r[...] + s1_r[...]
        dg = jnp.maximum(d0_r[...] + d1_r[...], 1.0)
        col = lax.broadcasted_iota(jnp.int32, (1, IN), 1)
        scale = jnp.where(col < 3, 1000.0, 1.0).astype(jnp.float32)
        aggs = (ssum / dg) * scale
        xs = x_r[...] * scale
        pre = (
            jnp.dot(aggs, wl_r[...], preferred_element_type=jnp.float32)
            + jnp.dot(xs, wr_r[...], preferred_element_type=jnp.float32)
            + b_r[...]
        )
        pre_r[...] = pre

        @pl.when(i == 0)
        def _():
            st_r[...] = jnp.zeros_like(st_r)

        st_r[0:1, :] += jnp.sum(pre, axis=0, keepdims=True)
        st_r[1:2, :] += jnp.sum(pre * pre, axis=0, keepdims=True)

    return pl.pallas_call(
        body,
        grid=(GRID,),
        in_specs=[
            pl.BlockSpec((BLK, IN), lambda i: (i, 0)),
            pl.BlockSpec((BLK, IN), lambda i: (i, 0)),
            pl.BlockSpec((BLK, 1), lambda i: (i, 0)),
            pl.BlockSpec((BLK, 1), lambda i: (i, 0)),
            pl.BlockSpec((BLK, IN), lambda i: (i, 0)),
            pl.BlockSpec((IN, H), lambda i: (0, 0)),
            pl.BlockSpec((IN, H), lambda i: (0, 0)),
            pl.BlockSpec((1, H), lambda i: (0, 0)),
        ],
        out_specs=[
            pl.BlockSpec((BLK, H), lambda i: (i, 0)),
            pl.BlockSpec((8, H), lambda i: (0, 0)),
        ],
        out_shape=[
            jax.ShapeDtypeStruct((N, H), jnp.float32),
            jax.ShapeDtypeStruct((8, H), jnp.float32),
        ],
    )(s0, s1, d0, d1, x, wl_t, wr_t, b1)


def _tc_bn_relu(pre, st, g, be):
    def body(pre_r, st_r, g_r, be_r, h_r):
        m = st_r[0:1, :] / N
        v = st_r[1:2, :] / N - m * m
        inv = lax.rsqrt(v + 1e-5) * g_r[...]
        h_r[...] = jnp.maximum((pre_r[...] - m) * inv + be_r[...], 0.0)

    return pl.pallas_call(
        body,
        grid=(GRID,),
        in_specs=[
            pl.BlockSpec((BLK, H), lambda i: (i, 0)),
            pl.BlockSpec((8, H), lambda i: (0, 0)),
            pl.BlockSpec((1, H), lambda i: (0, 0)),
            pl.BlockSpec((1, H), lambda i: (0, 0)),
        ],
        out_specs=pl.BlockSpec((BLK, H), lambda i: (i, 0)),
        out_shape=jax.ShapeDtypeStruct((N, H), jnp.float32),
    )(pre, st, g, be)


def _tc_layer2_pre(agg2, d0, d1, h1, wl_t, wr_t, b2):
    def body(agg_r, d0_r, d1_r, h_r, wl_r, wr_r, b_r, pre_r, st_r):
        i = pl.program_id(0)
        dg = jnp.maximum(d0_r[...] + d1_r[...], 1.0)
        agg = agg_r[...] / dg
        pre = (
            jnp.dot(agg, wl_r[...], preferred_element_type=jnp.float32)
            + jnp.dot(h_r[...], wr_r[...], preferred_element_type=jnp.float32)
            + b_r[...]
        )
        pre_r[...] = pre

        @pl.when(i == 0)
        def _():
            st_r[...] = jnp.zeros_like(st_r)

        st_r[0:1, :] += jnp.sum(pre, axis=0, keepdims=True)
        st_r[1:2, :] += jnp.sum(pre * pre, axis=0, keepdims=True)

    return pl.pallas_call(
        body,
        grid=(GRID,),
        in_specs=[
            pl.BlockSpec((BLK, H), lambda i: (i, 0)),
            pl.BlockSpec((BLK, 1), lambda i: (i, 0)),
            pl.BlockSpec((BLK, 1), lambda i: (i, 0)),
            pl.BlockSpec((BLK, H), lambda i: (i, 0)),
            pl.BlockSpec((H, H), lambda i: (0, 0)),
            pl.BlockSpec((H, H), lambda i: (0, 0)),
            pl.BlockSpec((1, H), lambda i: (0, 0)),
        ],
        out_specs=[
            pl.BlockSpec((BLK, H), lambda i: (i, 0)),
            pl.BlockSpec((8, H), lambda i: (0, 0)),
        ],
        out_shape=[
            jax.ShapeDtypeStruct((N, H), jnp.float32),
            jax.ShapeDtypeStruct((8, H), jnp.float32),
        ],
    )(agg2, d0, d1, h1, wl_t, wr_t, b2)


def _tc_final(pre2, st, g, be, wlin_t, blin):
    def body(pre_r, st_r, g_r, be_r, w_r, b_r, out_r):
        m = st_r[0:1, :] / N
        v = st_r[1:2, :] / N - m * m
        inv = lax.rsqrt(v + 1e-5) * g_r[...]
        h = jnp.maximum((pre_r[...] - m) * inv + be_r[...], 0.0)
        out_r[...] = (
            jnp.dot(h, w_r[...], preferred_element_type=jnp.float32) + b_r[...]
        )

    return pl.pallas_call(
        body,
        grid=(GRID,),
        in_specs=[
            pl.BlockSpec((BLK, H), lambda i: (i, 0)),
            pl.BlockSpec((8, H), lambda i: (0, 0)),
            pl.BlockSpec((1, H), lambda i: (0, 0)),
            pl.BlockSpec((1, H), lambda i: (0, 0)),
            pl.BlockSpec((H, OUT), lambda i: (0, 0)),
            pl.BlockSpec((1, OUT), lambda i: (0, 0)),
        ],
        out_specs=pl.BlockSpec((BLK, OUT), lambda i: (i, 0)),
        out_shape=jax.ShapeDtypeStruct((N, OUT), jnp.float32),
    )(pre2, st, g, be, wlin_t, blin)


def kernel(x, edge_index, W1_l, W1_r, b1, g1, be1, W2_l, W2_r, b2, g2, be2,
           W_lin, b_lin):
    src = edge_index[0]
    dst = edge_index[1]

    deg = jnp.zeros((N,), jnp.float32).at[dst].add(1.0)
    agg1 = jnp.zeros((N, IN), jnp.float32).at[dst].add(jnp.take(x, src, axis=0))
    d0 = deg[:, None]
    d1 = jnp.zeros_like(d0)
    s1 = jnp.zeros_like(agg1)

    pre1, st1 = _tc_layer1_pre(agg1, s1, d0, d1, x, W1_l.T, W1_r.T, b1[None])
    h1 = _tc_bn_relu(pre1, st1, g1[None], be1[None])

    agg2 = jnp.zeros((N, H), jnp.float32).at[dst].add(jnp.take(h1, src, axis=0))

    pre2, st2 = _tc_layer2_pre(agg2, d0, d1, h1, W2_l.T, W2_r.T, b2[None])
    return _tc_final(pre2, st2, g2[None], be2[None], W_lin.T, b_lin[None])
